# Initial kernel scaffold; baseline (speedup 1.0000x reference)
import jax
import jax.numpy as jnp
from jax.experimental import pallas as pl


def kernel(x, edge_index, W1, b1, s1W, s1b, g1, be1, m1, v1, W2, b2, s2W, s2b, skW, skb, g2, be2, m2, v2):
    raise NotImplementedError("write your pallas kernel here")


# speedup vs baseline: 13.2560x; 13.2560x over previous
import functools

import jax
import jax.numpy as jnp
from jax import lax
from jax.experimental import pallas as pl
from jax.experimental.pallas import tpu as pltpu
from jax.experimental.pallas import tpu_sc as plsc

_NC = 2
_NS = 16
_BLK = 128


def _sc_mesh():
    return plsc.VectorSubcoreMesh(core_axis_name="c", subcore_axis_name="s")


def _make_deg_kernel(nb, nz, d):
    rows_per_tile = nz // _NS

    @functools.partial(
        pl.kernel,
        out_type=jax.ShapeDtypeStruct((_NC, nz), jnp.float32),
        mesh=_sc_mesh(),
        scratch_types=[
            pltpu.VMEM((nb, _BLK), jnp.int32),
            pltpu.VMEM((_BLK,), jnp.float32),
            pltpu.VMEM_SHARED((nz,), jnp.float32),
        ],
    )
    def deg_kernel(dstp_hbm, ones_hbm, zeros_hbm, out_hbm, dsti_v, ones_v, acc_sh):
        c = lax.axis_index("c")
        s = lax.axis_index("s")
        row0 = s * rows_per_tile
        pltpu.sync_copy(zeros_hbm, acc_sh.at[pl.ds(row0, rows_per_tile)])
        pltpu.sync_copy(ones_hbm, ones_v)
        pltpu.sync_copy(dstp_hbm.at[c, s], dsti_v)
        plsc.subcore_barrier()

        def body(j, carry):
            pltpu.sync_copy(ones_v, acc_sh.at[dsti_v.at[j]], add=True)
            return carry

        lax.fori_loop(0, nb, body, 0)
        plsc.subcore_barrier()
        pltpu.sync_copy(acc_sh.at[pl.ds(row0, rows_per_tile)],
                        out_hbm.at[c, pl.ds(row0, rows_per_tile)])

    return deg_kernel


def _make_scatter_kernel(n, nb, nz, d):
    rows_per_tile = nz // _NS

    @functools.partial(
        pl.kernel,
        out_type=jax.ShapeDtypeStruct((_NC, nz, d), jnp.float32),
        mesh=_sc_mesh(),
        scratch_types=[
            pltpu.VMEM((2, 2, _BLK), jnp.int32),
            pltpu.VMEM((2, _BLK, d), jnp.float32),
            pltpu.VMEM_SHARED((nz, d), jnp.float32),
            pltpu.SemaphoreType.DMA((2,)),
            pltpu.SemaphoreType.DMA((2,)),
        ],
    )
    def scatter_kernel(y_hbm, idxp_hbm, zrows_hbm, out_hbm,
                       idx_v, rows_v, acc_sh, gsems, isems):
        c = lax.axis_index("c")
        s = lax.axis_index("s")
        row0 = s * rows_per_tile
        pltpu.sync_copy(zrows_hbm, acc_sh.at[pl.ds(row0, rows_per_tile)])
        plsc.subcore_barrier()

        pltpu.sync_copy(idxp_hbm.at[c, s, 0], idx_v.at[0])
        pltpu.async_copy(y_hbm.at[idx_v.at[0, 0]], rows_v.at[0], gsems.at[0])
        if nb > 1:
            pltpu.async_copy(idxp_hbm.at[c, s, 1], idx_v.at[1], isems.at[1])

        def body(j, carry):
            p = lax.rem(j, 2)
            q = 1 - p

            @pl.when(j + 1 < nb)
            def _():
                pltpu.make_async_copy(idxp_hbm.at[c, s, 0], idx_v.at[q],
                                      isems.at[q]).wait()
                pltpu.async_copy(y_hbm.at[idx_v.at[q, 0]], rows_v.at[q],
                                 gsems.at[q])

            pltpu.make_async_copy(y_hbm.at[idx_v.at[p, 0]], rows_v.at[p],
                                  gsems.at[p]).wait()
            pltpu.sync_copy(rows_v.at[p], acc_sh.at[idx_v.at[p, 1]], add=True)

            @pl.when(j + 2 < nb)
            def _():
                pltpu.async_copy(idxp_hbm.at[c, s, j + 2], idx_v.at[p],
                                 isems.at[p])

            return carry

        lax.fori_loop(0, nb, body, 0)
        plsc.subcore_barrier()
        pltpu.sync_copy(acc_sh.at[pl.ds(row0, rows_per_tile)],
                        out_hbm.at[c, pl.ds(row0, rows_per_tile)])

    return scatter_kernel


def _dot(a, b):
    return lax.dot_general(a, b, (((1,), (0,)), ((), ())),
                           precision=lax.Precision.HIGHEST,
                           preferred_element_type=jnp.float32)


def _leaky(x):
    return jnp.where(x >= 0, x, 0.01 * x)


def _row_specs(bn, d, n_full, n_one):
    full = [pl.BlockSpec((bn, d), lambda i: (i, 0)) for _ in range(n_full)]
    one = [pl.BlockSpec((1, d), lambda i: (0, 0)) for _ in range(n_one)]
    return full, one


def kernel(x, edge_index, W1, b1, s1W, s1b, g1, be1, m1, v1,
           W2, b2, s2W, s2b, skW, skb, g2, be2, m2, v2):
    n, d = x.shape
    e = edge_index.shape[1]
    nw = _NC * _NS

    ept = -(-e // nw)
    nb = -(-ept // _BLK)
    ep_pad = nb * _BLK
    nz = -(-(n + 1) // (_NS * 16)) * (_NS * 16)
    e_pad = ep_pad * nw
    src = jnp.pad(edge_index[0], (0, e_pad - e), constant_values=0)
    dst = jnp.pad(edge_index[1], (0, e_pad - e), constant_values=n)
    srcp = src.reshape(_NC, _NS, nb, _BLK)
    dstp = dst.reshape(_NC, _NS, nb, _BLK)
    idxp = jnp.stack([srcp, dstp], axis=3)
    ones128 = jnp.ones((_BLK,), jnp.float32)
    zeros1 = jnp.zeros((nz // _NS,), jnp.float32)
    zrows = jnp.zeros((nz // _NS, d), jnp.float32)

    deg_kernel = _make_deg_kernel(nb, nz, d)
    scatter_kernel = _make_scatter_kernel(n, nb, nz, d)

    degp = deg_kernel(dstp, ones128, zeros1)
    dega = degp[0, :n].reshape(n, 1)
    degb = degp[1, :n].reshape(n, 1)

    bn_rows = 2000
    grid = (n // bn_rows,)

    def tc_a_body(x_ref, w_ref, da_ref, db_ref, y_ref, dinv_ref):
        deg = da_ref[...] + db_ref[...] + 1.0
        dinv = lax.rsqrt(deg)
        dinv_ref[...] = dinv
        y_ref[...] = dinv * _dot(x_ref[...], w_ref[...])

    row_spec = pl.BlockSpec((bn_rows, d), lambda i: (i, 0))
    col_spec = pl.BlockSpec((bn_rows, 1), lambda i: (i, 0))
    w_spec = pl.BlockSpec((d, d), lambda i: (0, 0))
    p_spec = pl.BlockSpec((1, d), lambda i: (0, 0))

    y1, dinv = pl.pallas_call(
        tc_a_body,
        grid=grid,
        in_specs=[row_spec, w_spec, col_spec, col_spec],
        out_specs=[row_spec, col_spec],
        out_shape=[jax.ShapeDtypeStruct((n, d), jnp.float32),
                   jax.ShapeDtypeStruct((n, 1), jnp.float32)],
    )(x, W1, dega, degb)

    z1p = scatter_kernel(y1, idxp, zrows)
    z1a = z1p[0, :n]
    z1b = z1p[1, :n]

    def tc_c_body(za_ref, zb_ref, y_ref, x_ref, dinv_ref,
                  b1_ref, s1W_ref, s1b_ref, g1_ref, be1_ref, m1_ref, v1_ref,
                  W2_ref, h3_ref, y2_ref):
        dinv = dinv_ref[...]
        zsum = za_ref[...] + zb_ref[...] + y_ref[...]
        h1 = jax.nn.relu(dinv * zsum + b1_ref[...] + x_ref[...])
        h2 = jax.nn.relu(_leaky(_dot(h1, s1W_ref[...]) + s1b_ref[...]) + h1)
        bn1 = (h2 - m1_ref[...]) * lax.rsqrt(v1_ref[...] + 1e-5) * g1_ref[...] \
            + be1_ref[...]
        h3 = jax.nn.relu(bn1 + h2)
        h3_ref[...] = h3
        y2_ref[...] = dinv * _dot(h3, W2_ref[...])

    h3, y2 = pl.pallas_call(
        tc_c_body,
        grid=grid,
        in_specs=[row_spec, row_spec, row_spec, row_spec, col_spec,
                  p_spec, w_spec, p_spec, p_spec, p_spec, p_spec, p_spec,
                  w_spec],
        out_specs=[row_spec, row_spec],
        out_shape=[jax.ShapeDtypeStruct((n, d), jnp.float32),
                   jax.ShapeDtypeStruct((n, d), jnp.float32)],
    )(z1a, z1b, y1, x, dinv,
      b1.reshape(1, d), s1W, s1b.reshape(1, d), g1.reshape(1, d),
      be1.reshape(1, d), m1.reshape(1, d), v1.reshape(1, d), W2)

    z2p = scatter_kernel(y2, idxp, zrows)
    z2a = z2p[0, :n]
    z2b = z2p[1, :n]

    def tc_d_body(za_ref, zb_ref, y_ref, h3_ref, dinv_ref,
                  b2_ref, s2W_ref, s2b_ref, skW_ref, skb_ref,
                  g2_ref, be2_ref, m2_ref, v2_ref, out_ref):
        dinv = dinv_ref[...]
        zsum = za_ref[...] + zb_ref[...] + y_ref[...]
        h4 = jax.nn.relu(dinv * zsum + b2_ref[...] + h3_ref[...])
        h5 = jax.nn.relu(_leaky(_dot(h4, s2W_ref[...]) + s2b_ref[...]) + h4)
        h6 = jax.nn.relu(h5 + _dot(h5, skW_ref[...]) + skb_ref[...])
        bn2 = (h6 - m2_ref[...]) * lax.rsqrt(v2_ref[...] + 1e-5) * g2_ref[...] \
            + be2_ref[...]
        out_ref[...] = jax.nn.relu(bn2 + h6)

    out = pl.pallas_call(
        tc_d_body,
        grid=grid,
        in_specs=[row_spec, row_spec, row_spec, row_spec, col_spec,
                  p_spec, w_spec, p_spec, w_spec, p_spec,
                  p_spec, p_spec, p_spec, p_spec],
        out_specs=row_spec,
        out_shape=jax.ShapeDtypeStruct((n, d), jnp.float32),
    )(z2a, z2b, y2, h3, dinv,
      b2.reshape(1, d), s2W, s2b.reshape(1, d), skW, skb.reshape(1, d),
      g2.reshape(1, d), be2.reshape(1, d), m2.reshape(1, d), v2.reshape(1, d))

    return out
